# baseline (speedup 1.0000x reference)
import functools

import jax
import jax.numpy as jnp
from jax import lax
from jax.experimental import pallas as pl
from jax.experimental.pallas import tpu as pltpu, tpu_sc as plsc

N = 5000
NPAD = 5120
NCHUNK = NPAD // 16
NCLS = 21
NFG = NCLS - 1
K = 100
KPAD = 128
NMS_THRESH = 0.3
SCORE_THRESH = 0.05
IMG_H = 600.0
IMG_W = 800.0
NEG = -1.0


def _prep_body(scores_ref, loc_ref, rois_ref, probs_ref, boxes_ref):
    scores = scores_ref[...]
    mx = jnp.max(scores, axis=0, keepdims=True)
    e = jnp.exp(scores - mx)
    denom = jnp.sum(e, axis=0, keepdims=True)
    col = lax.broadcasted_iota(jnp.int32, (1, NPAD), 1)
    row_valid = col < N
    probs_ref[...] = jnp.where(row_valid, e / denom, 0.0)

    src_h = rois_ref[2:3, :] - rois_ref[0:1, :]
    src_w = rois_ref[3:4, :] - rois_ref[1:2, :]
    ctr_y = rois_ref[0:1, :] + 0.5 * src_h
    ctr_x = rois_ref[1:2, :] + 0.5 * src_w

    dy = loc_ref[0] * 0.1
    dx = loc_ref[1] * 0.1
    dh = loc_ref[2] * 0.2
    dw = loc_ref[3] * 0.2
    h = jnp.exp(dh) * src_h
    w = jnp.exp(dw) * src_w
    cy = dy * src_h + ctr_y
    cx = dx * src_w + ctr_x
    boxes_ref[0] = jnp.clip(cy - 0.5 * h, 0.0, IMG_H)
    boxes_ref[1] = jnp.clip(cx - 0.5 * w, 0.0, IMG_W)
    boxes_ref[2] = jnp.clip(cy + 0.5 * h, 0.0, IMG_H)
    boxes_ref[3] = jnp.clip(cx + 0.5 * w, 0.0, IMG_W)


def _nms_body(probs_hbm, boxes_hbm, ob_hbm, os_hbm, ol_hbm,
              score_v, boxes_v, cm_v, cm2_v, sel_v, ob_v, os_v, ol_v, dsem):
    cidx = lax.axis_index("c")
    sidx = lax.axis_index("s")
    wid = sidx * 2 + cidx

    @pl.when(wid < NFG)
    def _work():
        cls = wid + 1
        h1 = pltpu.async_copy(probs_hbm.at[cls], score_v, dsem)
        h2 = pltpu.async_copy(boxes_hbm.at[0, cls], boxes_v.at[0], dsem)
        h3 = pltpu.async_copy(boxes_hbm.at[1, cls], boxes_v.at[1], dsem)
        h4 = pltpu.async_copy(boxes_hbm.at[2, cls], boxes_v.at[2], dsem)
        h5 = pltpu.async_copy(boxes_hbm.at[3, cls], boxes_v.at[3], dsem)
        lane = lax.iota(jnp.int32, 16)
        zf16 = jnp.zeros((16,), jnp.float32)
        negv = jnp.full((16,), NEG, jnp.float32)

        @plsc.parallel_loop(0, (KPAD * 4) // 16, unroll=4)
        def _zob(i):
            ob_v[pl.ds(i * 16, 16)] = zf16

        @plsc.parallel_loop(0, KPAD // 16, unroll=2)
        def _zos(i):
            os_v[pl.ds(i * 16, 16)] = zf16
            sel_v[0, pl.ds(i * 16, 16)] = zf16
            sel_v[1, pl.ds(i * 16, 16)] = zf16
            sel_v[2, pl.ds(i * 16, 16)] = zf16
            sel_v[3, pl.ds(i * 16, 16)] = zf16
            sel_v[4, pl.ds(i * 16, 16)] = zf16

        @plsc.parallel_loop(0, (NCHUNK + 16) // 16, unroll=4)
        def _zcm(i):
            cm_v[pl.ds(i * 16, 16)] = negv

        cm2_v[pl.ds(0, 16)] = negv
        cm2_v[pl.ds(16, 16)] = negv
        h1.wait()
        h2.wait()
        h3.wait()
        h4.wait()
        h5.wait()

        def cmb(i, _):
            for k in range(8):
                j = i * 8 + k
                p = score_v[pl.ds(j * 16, 16)]
                s = jnp.where(p > SCORE_THRESH, p, NEG)
                score_v[pl.ds(j * 16, 16)] = s
                plsc.store_scatter(cm_v, [jnp.full((16,), j, jnp.int32)],
                                   jnp.full((16,), jnp.max(s), jnp.float32),
                                   mask=lane == 0)
            return 0

        lax.fori_loop(0, NCHUNK // 8, cmb, 0)

        def cm2b(g, _):
            cmg = cm_v[pl.ds(g * 16, 16)]
            plsc.store_scatter(cm2_v, [jnp.full((16,), g, jnp.int32)],
                               jnp.full((16,), jnp.max(cmg), jnp.float32),
                               mask=lane == 0)
            return 0

        lax.fori_loop(0, NCHUNK // 16, cm2b, 0)

        m0 = jnp.max(jnp.maximum(cm2_v[pl.ds(0, 16)], cm2_v[pl.ds(16, 16)]))

        def cond(carry):
            ns, m = carry
            return (ns < K) & (m > 0.0)

        def body(carry):
            ns, m = carry
            c0 = cm2_v[pl.ds(0, 16)]
            c1 = cm2_v[pl.ds(16, 16)]
            f0 = plsc.all_reduce_ffs(c0 >= m)[0]
            f1 = plsc.all_reduce_ffs(c1 >= m)[0]
            g = jnp.where(f0 < 16, f0, 16 + f1)
            cmg = cm_v[pl.ds(g * 16, 16)]
            jin = plsc.all_reduce_ffs(cmg >= m)[0]
            j = g * 16 + jin
            s = score_v[pl.ds(j * 16, 16)]
            lsel = plsc.all_reduce_ffs(s >= m)[0]
            idx = j * 16 + lsel
            s2 = jnp.where(lane == lsel, NEG, s)
            score_v[pl.ds(j * 16, 16)] = s2
            cmj = jnp.max(s2)
            cmg2 = jnp.where(lane == jin, cmj, cmg)
            cm_v[pl.ds(g * 16, 16)] = cmg2
            maxg = jnp.max(cmg2)
            plsc.store_scatter(cm2_v, [jnp.full((16,), g, jnp.int32)],
                               jnp.full((16,), maxg, jnp.float32),
                               mask=lane == 0)
            c0n = jnp.where(lane == g, maxg, c0)
            c1n = jnp.where(lane + 16 == g, maxg, c1)
            m2 = jnp.max(jnp.maximum(c0n, c1n))

            idxv = jnp.full((16,), idx, jnp.int32)
            by1 = plsc.load_gather(boxes_v, [jnp.zeros((16,), jnp.int32), idxv])
            bx1 = plsc.load_gather(boxes_v, [jnp.full((16,), 1, jnp.int32), idxv])
            by2 = plsc.load_gather(boxes_v, [jnp.full((16,), 2, jnp.int32), idxv])
            bx2 = plsc.load_gather(boxes_v, [jnp.full((16,), 3, jnp.int32), idxv])
            ca = jnp.maximum(by2 - by1, 0.0) * jnp.maximum(bx2 - bx1, 0.0)

            def tchk(tb, mi):
                sy1 = sel_v[0, pl.ds(tb, 16)]
                sx1 = sel_v[1, pl.ds(tb, 16)]
                sy2 = sel_v[2, pl.ds(tb, 16)]
                sx2 = sel_v[3, pl.ds(tb, 16)]
                sa = sel_v[4, pl.ds(tb, 16)]
                tl_y = jnp.maximum(sy1, by1)
                tl_x = jnp.maximum(sx1, bx1)
                br_y = jnp.minimum(sy2, by2)
                br_x = jnp.minimum(sx2, bx2)
                wh_y = jnp.maximum(br_y - tl_y, 0.0)
                wh_x = jnp.maximum(br_x - tl_x, 0.0)
                inter = wh_y * wh_x
                iou = inter / (sa + ca - inter + 1e-9)
                return jnp.maximum(mi, iou)

            maxiou_v = plsc.parallel_loop(
                0, ((ns + 15) // 16) * 16, 16, unroll=4, carry=zf16)(tchk)
            keep = jnp.max(maxiou_v) <= NMS_THRESH

            @pl.when(keep)
            def _acc():
                nsv = jnp.full((16,), ns, jnp.int32)
                lane0 = lane == 0
                zi0 = jnp.zeros((16,), jnp.int32)
                plsc.store_scatter(sel_v, [zi0, nsv], by1, mask=lane0)
                plsc.store_scatter(sel_v, [zi0 + 1, nsv], bx1, mask=lane0)
                plsc.store_scatter(sel_v, [zi0 + 2, nsv], by2, mask=lane0)
                plsc.store_scatter(sel_v, [zi0 + 3, nsv], bx2, mask=lane0)
                plsc.store_scatter(sel_v, [zi0 + 4, nsv], ca, mask=lane0)
                boxvec = jnp.where(lane == 0, by1,
                                   jnp.where(lane == 1, bx1,
                                             jnp.where(lane == 2, by2, bx2)))
                plsc.store_scatter(ob_v, [ns * 4 + lane], boxvec,
                                   mask=lane < 4)
                plsc.store_scatter(os_v, [nsv],
                                   jnp.full((16,), m, jnp.float32),
                                   mask=lane0)

            ns2 = jnp.where(keep, ns + 1, ns)
            return ns2, m2

        kfin, _ = lax.while_loop(cond, body, (jnp.int32(0), m0))

        def lfill(i, _):
            base = i * 16
            ol_v[pl.ds(base, 16)] = jnp.where(base + lane < kfin, wid, -1)
            return 0

        lax.fori_loop(0, KPAD // 16, lfill, 0)

        pltpu.sync_copy(ob_v, ob_hbm.at[wid])
        pltpu.sync_copy(os_v, os_hbm.at[wid])
        pltpu.sync_copy(ol_v, ol_hbm.at[wid])


@jax.jit
def kernel(roi_cls_loc, roi_scores, rois):
    pad = NPAD - N
    scores_t = jnp.pad(roi_scores, ((0, pad), (0, 0))).T
    loc_t = jnp.transpose(
        jnp.pad(roi_cls_loc.reshape(N, NCLS, 4), ((0, pad), (0, 0), (0, 0))),
        (2, 1, 0))
    rois_t = jnp.pad(rois, ((0, pad), (0, 0))).T

    probs, boxes = pl.pallas_call(
        _prep_body,
        out_shape=[
            jax.ShapeDtypeStruct((NCLS, NPAD), jnp.float32),
            jax.ShapeDtypeStruct((4, NCLS, NPAD), jnp.float32),
        ],
    )(scores_t, loc_t, rois_t)

    nms = pl.kernel(
        _nms_body,
        out_type=[
            jax.ShapeDtypeStruct((NFG, KPAD * 4), jnp.float32),
            jax.ShapeDtypeStruct((NFG, KPAD), jnp.float32),
            jax.ShapeDtypeStruct((NFG, KPAD), jnp.int32),
        ],
        mesh=plsc.VectorSubcoreMesh(core_axis_name="c", subcore_axis_name="s"),
        compiler_params=pltpu.CompilerParams(needs_layout_passes=False),
        scratch_types=[
            pltpu.VMEM((NPAD,), jnp.float32),
            pltpu.VMEM((4, NPAD), jnp.float32),
            pltpu.VMEM((NCHUNK + 16,), jnp.float32),
            pltpu.VMEM((32,), jnp.float32),
            pltpu.VMEM((5, KPAD), jnp.float32),
            pltpu.VMEM((KPAD * 4,), jnp.float32),
            pltpu.VMEM((KPAD,), jnp.float32),
            pltpu.VMEM((KPAD,), jnp.int32),
            pltpu.SemaphoreType.DMA,
        ],
    )
    ob, os_, ol = nms(probs, boxes)

    out_boxes = ob.reshape(NFG, KPAD, 4)[:, :K, :].reshape(-1, 4)
    out_scores = os_[:, :K].reshape(-1)
    out_labels = ol[:, :K].reshape(-1)
    return out_boxes, out_labels, out_scores


# speedup vs baseline: 1.0153x; 1.0153x over previous
import functools

import jax
import jax.numpy as jnp
from jax import lax
from jax.experimental import pallas as pl
from jax.experimental.pallas import tpu as pltpu, tpu_sc as plsc

N = 5000
NPAD = 5120
NCHUNK = NPAD // 16
NCLS = 21
NFG = NCLS - 1
K = 100
KPAD = 128
NMS_THRESH = 0.3
SCORE_THRESH = 0.05
IMG_H = 600.0
IMG_W = 800.0
NEG = -1.0


def _prep_body(scores_ref, loc_ref, rois_ref, probs_ref, boxes_ref):
    scores = scores_ref[...]
    mx = jnp.max(scores, axis=0, keepdims=True)
    e = jnp.exp(scores - mx)
    denom = jnp.sum(e, axis=0, keepdims=True)
    col = lax.broadcasted_iota(jnp.int32, (1, NPAD), 1)
    row_valid = col < N
    probs_ref[...] = jnp.where(row_valid, e / denom, 0.0)

    src_h = rois_ref[2:3, :] - rois_ref[0:1, :]
    src_w = rois_ref[3:4, :] - rois_ref[1:2, :]
    ctr_y = rois_ref[0:1, :] + 0.5 * src_h
    ctr_x = rois_ref[1:2, :] + 0.5 * src_w

    dy = loc_ref[0] * 0.1
    dx = loc_ref[1] * 0.1
    dh = loc_ref[2] * 0.2
    dw = loc_ref[3] * 0.2
    h = jnp.exp(dh) * src_h
    w = jnp.exp(dw) * src_w
    cy = dy * src_h + ctr_y
    cx = dx * src_w + ctr_x
    boxes_ref[0] = jnp.clip(cy - 0.5 * h, 0.0, IMG_H)
    boxes_ref[1] = jnp.clip(cx - 0.5 * w, 0.0, IMG_W)
    boxes_ref[2] = jnp.clip(cy + 0.5 * h, 0.0, IMG_H)
    boxes_ref[3] = jnp.clip(cx + 0.5 * w, 0.0, IMG_W)


def _nms_body(probs_hbm, boxes_hbm, ob_hbm, os_hbm, ol_hbm,
              score_v, boxes_v, cm_v, cm2_v, sel_v, ob_v, os_v, ol_v, dsem):
    cidx = lax.axis_index("c")
    sidx = lax.axis_index("s")
    wid = sidx * 2 + cidx

    @pl.when(wid < NFG)
    def _work():
        cls = wid + 1
        h1 = pltpu.async_copy(probs_hbm.at[cls], score_v, dsem)
        h2 = pltpu.async_copy(boxes_hbm.at[0, cls], boxes_v.at[0], dsem)
        h3 = pltpu.async_copy(boxes_hbm.at[1, cls], boxes_v.at[1], dsem)
        h4 = pltpu.async_copy(boxes_hbm.at[2, cls], boxes_v.at[2], dsem)
        h5 = pltpu.async_copy(boxes_hbm.at[3, cls], boxes_v.at[3], dsem)
        lane = lax.iota(jnp.int32, 16)
        zf16 = jnp.zeros((16,), jnp.float32)
        negv = jnp.full((16,), NEG, jnp.float32)

        @plsc.parallel_loop(0, (KPAD * 4) // 16, unroll=4)
        def _zob(i):
            ob_v[pl.ds(i * 16, 16)] = zf16

        @plsc.parallel_loop(0, KPAD // 16, unroll=2)
        def _zos(i):
            os_v[pl.ds(i * 16, 16)] = zf16
            sel_v[0, pl.ds(i * 16, 16)] = zf16
            sel_v[1, pl.ds(i * 16, 16)] = zf16
            sel_v[2, pl.ds(i * 16, 16)] = zf16
            sel_v[3, pl.ds(i * 16, 16)] = zf16
            sel_v[4, pl.ds(i * 16, 16)] = zf16

        @plsc.parallel_loop(0, (NCHUNK + 16) // 16, unroll=4)
        def _zcm(i):
            cm_v[pl.ds(i * 16, 16)] = negv

        cm2_v[pl.ds(0, 16)] = negv
        cm2_v[pl.ds(16, 16)] = negv
        h1.wait()
        h2.wait()
        h3.wait()
        h4.wait()
        h5.wait()

        def cmb(i, _):
            for k in range(4):
                j = i * 4 + k
                p = score_v[pl.ds(j * 16, 16)]
                s = jnp.where(p > SCORE_THRESH, p, NEG)
                score_v[pl.ds(j * 16, 16)] = s
                plsc.store_scatter(cm_v, [jnp.full((16,), j, jnp.int32)],
                                   jnp.full((16,), jnp.max(s), jnp.float32),
                                   mask=lane == 0)
            return 0

        lax.fori_loop(0, NCHUNK // 4, cmb, 0)

        def cm2b(g, _):
            cmg = cm_v[pl.ds(g * 16, 16)]
            plsc.store_scatter(cm2_v, [jnp.full((16,), g, jnp.int32)],
                               jnp.full((16,), jnp.max(cmg), jnp.float32),
                               mask=lane == 0)
            return 0

        lax.fori_loop(0, NCHUNK // 16, cm2b, 0)

        m0 = jnp.max(jnp.maximum(cm2_v[pl.ds(0, 16)], cm2_v[pl.ds(16, 16)]))

        def cond(carry):
            ns, m = carry
            return (ns < K) & (m > 0.0)

        def body(carry):
            ns, m = carry
            c0 = cm2_v[pl.ds(0, 16)]
            c1 = cm2_v[pl.ds(16, 16)]
            f0 = plsc.all_reduce_ffs(c0 >= m)[0]
            f1 = plsc.all_reduce_ffs(c1 >= m)[0]
            g = jnp.where(f0 < 16, f0, 16 + f1)
            cmg = cm_v[pl.ds(g * 16, 16)]
            jin = plsc.all_reduce_ffs(cmg >= m)[0]
            j = g * 16 + jin
            s = score_v[pl.ds(j * 16, 16)]
            lsel = plsc.all_reduce_ffs(s >= m)[0]
            idx = j * 16 + lsel
            s2 = jnp.where(lane == lsel, NEG, s)
            score_v[pl.ds(j * 16, 16)] = s2
            cmj = jnp.max(s2)
            cmg2 = jnp.where(lane == jin, cmj, cmg)
            cm_v[pl.ds(g * 16, 16)] = cmg2
            maxg = jnp.max(cmg2)
            plsc.store_scatter(cm2_v, [jnp.full((16,), g, jnp.int32)],
                               jnp.full((16,), maxg, jnp.float32),
                               mask=lane == 0)
            c0n = jnp.where(lane == g, maxg, c0)
            c1n = jnp.where(lane + 16 == g, maxg, c1)
            m2 = jnp.max(jnp.maximum(c0n, c1n))

            idxv = jnp.full((16,), idx, jnp.int32)
            by1 = plsc.load_gather(boxes_v, [jnp.zeros((16,), jnp.int32), idxv])
            bx1 = plsc.load_gather(boxes_v, [jnp.full((16,), 1, jnp.int32), idxv])
            by2 = plsc.load_gather(boxes_v, [jnp.full((16,), 2, jnp.int32), idxv])
            bx2 = plsc.load_gather(boxes_v, [jnp.full((16,), 3, jnp.int32), idxv])
            ca = jnp.maximum(by2 - by1, 0.0) * jnp.maximum(bx2 - bx1, 0.0)

            def tchk(tb, mi):
                sy1 = sel_v[0, pl.ds(tb, 16)]
                sx1 = sel_v[1, pl.ds(tb, 16)]
                sy2 = sel_v[2, pl.ds(tb, 16)]
                sx2 = sel_v[3, pl.ds(tb, 16)]
                sa = sel_v[4, pl.ds(tb, 16)]
                tl_y = jnp.maximum(sy1, by1)
                tl_x = jnp.maximum(sx1, bx1)
                br_y = jnp.minimum(sy2, by2)
                br_x = jnp.minimum(sx2, bx2)
                wh_y = jnp.maximum(br_y - tl_y, 0.0)
                wh_x = jnp.maximum(br_x - tl_x, 0.0)
                inter = wh_y * wh_x
                iou = inter / (sa + ca - inter + 1e-9)
                return jnp.maximum(mi, iou)

            maxiou_v = plsc.parallel_loop(
                0, ((ns + 15) // 16) * 16, 16, unroll=2, carry=zf16)(tchk)
            keep = jnp.max(maxiou_v) <= NMS_THRESH

            @pl.when(keep)
            def _acc():
                nsv = jnp.full((16,), ns, jnp.int32)
                lane0 = lane == 0
                zi0 = jnp.zeros((16,), jnp.int32)
                plsc.store_scatter(sel_v, [zi0, nsv], by1, mask=lane0)
                plsc.store_scatter(sel_v, [zi0 + 1, nsv], bx1, mask=lane0)
                plsc.store_scatter(sel_v, [zi0 + 2, nsv], by2, mask=lane0)
                plsc.store_scatter(sel_v, [zi0 + 3, nsv], bx2, mask=lane0)
                plsc.store_scatter(sel_v, [zi0 + 4, nsv], ca, mask=lane0)
                boxvec = jnp.where(lane == 0, by1,
                                   jnp.where(lane == 1, bx1,
                                             jnp.where(lane == 2, by2, bx2)))
                plsc.store_scatter(ob_v, [ns * 4 + lane], boxvec,
                                   mask=lane < 4)
                plsc.store_scatter(os_v, [nsv],
                                   jnp.full((16,), m, jnp.float32),
                                   mask=lane0)

            ns2 = jnp.where(keep, ns + 1, ns)
            return ns2, m2

        kfin, _ = lax.while_loop(cond, body, (jnp.int32(0), m0))

        def lfill(i, _):
            base = i * 16
            ol_v[pl.ds(base, 16)] = jnp.where(base + lane < kfin, wid, -1)
            return 0

        lax.fori_loop(0, KPAD // 16, lfill, 0)

        pltpu.sync_copy(ob_v, ob_hbm.at[wid])
        pltpu.sync_copy(os_v, os_hbm.at[wid])
        pltpu.sync_copy(ol_v, ol_hbm.at[wid])


@jax.jit
def kernel(roi_cls_loc, roi_scores, rois):
    pad = NPAD - N
    scores_t = jnp.pad(roi_scores, ((0, pad), (0, 0))).T
    loc_t = jnp.transpose(
        jnp.pad(roi_cls_loc.reshape(N, NCLS, 4), ((0, pad), (0, 0), (0, 0))),
        (2, 1, 0))
    rois_t = jnp.pad(rois, ((0, pad), (0, 0))).T

    probs, boxes = pl.pallas_call(
        _prep_body,
        out_shape=[
            jax.ShapeDtypeStruct((NCLS, NPAD), jnp.float32),
            jax.ShapeDtypeStruct((4, NCLS, NPAD), jnp.float32),
        ],
    )(scores_t, loc_t, rois_t)

    nms = pl.kernel(
        _nms_body,
        out_type=[
            jax.ShapeDtypeStruct((NFG, KPAD * 4), jnp.float32),
            jax.ShapeDtypeStruct((NFG, KPAD), jnp.float32),
            jax.ShapeDtypeStruct((NFG, KPAD), jnp.int32),
        ],
        mesh=plsc.VectorSubcoreMesh(core_axis_name="c", subcore_axis_name="s"),
        compiler_params=pltpu.CompilerParams(needs_layout_passes=False),
        scratch_types=[
            pltpu.VMEM((NPAD,), jnp.float32),
            pltpu.VMEM((4, NPAD), jnp.float32),
            pltpu.VMEM((NCHUNK + 16,), jnp.float32),
            pltpu.VMEM((32,), jnp.float32),
            pltpu.VMEM((5, KPAD), jnp.float32),
            pltpu.VMEM((KPAD * 4,), jnp.float32),
            pltpu.VMEM((KPAD,), jnp.float32),
            pltpu.VMEM((KPAD,), jnp.int32),
            pltpu.SemaphoreType.DMA,
        ],
    )
    ob, os_, ol = nms(probs, boxes)

    out_boxes = ob.reshape(NFG, KPAD, 4)[:, :K, :].reshape(-1, 4)
    out_scores = os_[:, :K].reshape(-1)
    out_labels = ol[:, :K].reshape(-1)
    return out_boxes, out_labels, out_scores
